# Initial kernel scaffold; baseline (speedup 1.0000x reference)
import jax
import jax.numpy as jnp
from jax.experimental import pallas as pl


def kernel(x, edge_index, edge_attr, W_edge, b_edge, W_node, b_node):
    raise NotImplementedError("write your pallas kernel here")


# speedup vs baseline: 5.9776x; 5.9776x over previous
import functools

import jax
import jax.numpy as jnp
from jax import lax
from jax.experimental import pallas as pl
from jax.experimental.pallas import tpu as pltpu
from jax.experimental.pallas import tpu_sc as plsc

_N = 10000
_E = 320000
_D = 128
_DE = 16
_EB = _E // 8

_NC = 2
_NS = 16
_NW = _NC * _NS
_EW = _E // _NW
_B = 1000
_NB = _EW // _B
_SB = 125
_JR = _B // _SB
_NP = 10240
_NPT = _NP // _NS


def _node_proj_body(x_ref, ws_ref, wr_ref, xs_ref, xr_ref):
    xv = x_ref[...]
    xs_ref[...] = jnp.dot(xv, ws_ref[...], preferred_element_type=jnp.float32)
    xr_ref[...] = jnp.dot(xv, wr_ref[...], preferred_element_type=jnp.float32)


def _edge_proj_body(ea_ref, bd_ref, b_ref, o_ref):
    o_ref[...] = jnp.dot(ea_ref[...], bd_ref[...],
                         preferred_element_type=jnp.float32) + b_ref[...]


def _node_mlp_body(x_ref, a0_ref, a1_ref, wt_ref, wb_ref, b_ref, o_ref):
    acc = jnp.dot(x_ref[...], wt_ref[...], preferred_element_type=jnp.float32)
    acc = acc + jnp.dot(a0_ref[...] + a1_ref[...], wb_ref[...],
                        preferred_element_type=jnp.float32)
    o_ref[...] = jnp.maximum(acc + b_ref[...], 0.0)


def _sc_edges_body(xs_hbm, xr_hbm, ea_hbm, s2_hbm, r2_hbm,
                   m_hbm, parts_hbm,
                   idx_s, idx_r, bufs, bufr, bufe, bufo, zbuf, agg_sh,
                   sem_g, sem_w):
    c = lax.axis_index("c")
    s = lax.axis_index("s")
    wid = c * _NS + s

    def _zero(i, carry):
        zbuf[i] = jnp.zeros((_DE,), jnp.float32)
        return carry
    lax.fori_loop(0, _NPT, _zero, 0)
    pltpu.sync_copy(zbuf, agg_sh.at[pl.ds(s * _NPT, _NPT)])
    plsc.subcore_barrier()

    e0 = wid * _EW

    def _block(b, carry):
        eb0 = e0 + b * _B
        row0 = wid * (_EW // _SB) + b * _JR
        pltpu.sync_copy(s2_hbm.at[pl.ds(row0, _JR)], idx_s)
        pltpu.sync_copy(r2_hbm.at[pl.ds(row0, _JR)], idx_r)
        pltpu.sync_copy(ea_hbm.at[pl.ds(eb0 // 8, _B // 8)], bufe)
        cps = []
        for j in range(_JR):
            cps.append(pltpu.async_copy(xs_hbm.at[idx_s.at[j]],
                                        bufs.at[pl.ds(j * _SB, _SB)], sem_g))
            cps.append(pltpu.async_copy(xr_hbm.at[idx_r.at[j]],
                                        bufr.at[pl.ds(j * _SB, _SB)], sem_g))
        for cp in cps:
            cp.wait()

        def _relu(i2, carry):
            for u in range(8):
                row = i2 * 8 + u
                val = jnp.maximum(
                    bufs[row] + bufr[row] + bufe[i2, pl.ds(u * _DE, _DE)], 0.0)
                bufs[row] = val
                bufo[i2, pl.ds(u * _DE, _DE)] = val
            return carry
        lax.fori_loop(0, _B // 8, _relu, 0)

        wcps = [pltpu.async_copy(bufo, m_hbm.at[pl.ds(eb0 // 8, _B // 8)],
                                 sem_w)]
        for j in range(_JR):
            pltpu.sync_copy(bufs.at[pl.ds(j * _SB, _SB)],
                            agg_sh.at[idx_r.at[j]], add=True)
        for cp in wcps:
            cp.wait()
        return carry

    lax.fori_loop(0, _NB, _block, 0)

    plsc.subcore_barrier()
    pltpu.sync_copy(agg_sh.at[pl.ds(s * _NPT, _NPT)], zbuf)
    pltpu.sync_copy(zbuf, parts_hbm.at[c, pl.ds(s * _NPT, _NPT)])


_sc_edges = functools.partial(
    pl.kernel,
    out_type=(jax.ShapeDtypeStruct((_EB, _D), jnp.float32),
              jax.ShapeDtypeStruct((_NC, _NP, _DE), jnp.float32)),
    mesh=plsc.VectorSubcoreMesh(core_axis_name="c", subcore_axis_name="s"),
    compiler_params=pltpu.CompilerParams(use_tc_tiling_on_sc=False),
    scratch_types=[
        pltpu.VMEM((_JR, _SB), jnp.int32),
        pltpu.VMEM((_JR, _SB), jnp.int32),
        pltpu.VMEM((_B, _DE), jnp.float32),
        pltpu.VMEM((_B, _DE), jnp.float32),
        pltpu.VMEM((_B // 8, _D), jnp.float32),
        pltpu.VMEM((_B // 8, _D), jnp.float32),
        pltpu.VMEM((_NPT, _DE), jnp.float32),
        pltpu.VMEM_SHARED((_NP, _DE), jnp.float32),
        pltpu.SemaphoreType.DMA,
        pltpu.SemaphoreType.DMA,
    ],
)(_sc_edges_body)


@jax.jit
def kernel(x, edge_index, edge_attr, W_edge, b_edge, W_node, b_node):
    senders = edge_index[0]
    receivers = edge_index[1]
    w_s = W_edge[:_D]
    w_r = W_edge[_D:2 * _D]
    w_a = W_edge[2 * _D:]

    xs, xr = pl.pallas_call(
        _node_proj_body,
        out_shape=(jax.ShapeDtypeStruct((_N, _DE), jnp.float32),
                   jax.ShapeDtypeStruct((_N, _DE), jnp.float32)),
    )(x, w_s, w_r)

    ea2 = edge_attr.reshape(_EB, _D)
    bd = jnp.kron(jnp.eye(8, dtype=jnp.float32), w_a)
    bt = jnp.tile(b_edge, 8).reshape(1, _D)
    ea = pl.pallas_call(
        _edge_proj_body,
        grid=(10,),
        in_specs=[pl.BlockSpec((_EB // 10, _D), lambda i: (i, 0)),
                  pl.BlockSpec((_D, _D), lambda i: (0, 0)),
                  pl.BlockSpec((1, _D), lambda i: (0, 0))],
        out_specs=pl.BlockSpec((_EB // 10, _D), lambda i: (i, 0)),
        out_shape=jax.ShapeDtypeStruct((_EB, _D), jnp.float32),
    )(ea2, bd, bt)

    s2 = senders.reshape(_E // _SB, _SB)
    r2 = receivers.reshape(_E // _SB, _SB)
    m_blocked, parts = _sc_edges(xs, xr, ea, s2, r2)
    m = m_blocked.reshape(_E, _DE)
    parts = parts[:, :_N]

    nodes = pl.pallas_call(
        _node_mlp_body,
        grid=(10,),
        in_specs=[pl.BlockSpec((_N // 10, _D), lambda i: (i, 0)),
                  pl.BlockSpec((_N // 10, _DE), lambda i: (i, 0)),
                  pl.BlockSpec((_N // 10, _DE), lambda i: (i, 0)),
                  pl.BlockSpec((_D, _D), lambda i: (0, 0)),
                  pl.BlockSpec((_DE, _D), lambda i: (0, 0)),
                  pl.BlockSpec((1, _D), lambda i: (0, 0))],
        out_specs=pl.BlockSpec((_N // 10, _D), lambda i: (i, 0)),
        out_shape=jax.ShapeDtypeStruct((_N, _D), jnp.float32),
    )(x, parts[0], parts[1], W_node[:_D], W_node[_D:], b_node.reshape(1, _D))

    return nodes, m
